# baseline (speedup 1.0000x reference)
import functools

import jax
import jax.numpy as jnp
import numpy as np
from jax import lax
from jax.experimental import pallas as pl
from jax.experimental.pallas import tpu as pltpu
from jax.experimental.pallas import tpu_sc as plsc

N_NODES = 10000
N_PAD = 10240
D = 128
E = 320000
L = 16
NC = 2
NS = 16
G = 64
E_PAD = 327680
GROUPS = E_PAD // G
GPT = GROUPS // (NC * NS)
SGC = 16
NSTG = GPT // SGC
NB = 4
NS2 = 2
RPT = N_PAD // NS
K1_G = 128
K1_GPT = (E_PAD // K1_G) // (NC * NS)

_PERM = np.empty((D,), dtype=np.int32)
for _p in range(D):
    _j, _r = _p // 32, _p % 32
    _PERM[_p] = 32 * _j + (_r // 2) + 16 * (_r % 2)

_sc_mesh = plsc.VectorSubcoreMesh(
    core_axis_name="c", subcore_axis_name="s", num_cores=NC, num_subcores=NS
)

_ZV = lambda: jnp.zeros((L,), jnp.float32)

_sc_params = pltpu.CompilerParams(needs_layout_passes=False)
_sc_params_sc_tiling = pltpu.CompilerParams(
    needs_layout_passes=False, use_tc_tiling_on_sc=False
)


@functools.partial(
    pl.kernel,
    out_type=jax.ShapeDtypeStruct((NC, N_PAD), jnp.float32),
    mesh=_sc_mesh,
    scratch_types=[
        pltpu.VMEM_SHARED((N_PAD,), jnp.float32),
        pltpu.VMEM((RPT,), jnp.float32),
        pltpu.VMEM((K1_GPT, K1_G), jnp.int32),
        pltpu.VMEM((K1_GPT, K1_G), jnp.float32),
    ],
    compiler_params=_sc_params,
)
def _deg_kernel(col_ref, w_ref, deg_out, deg_sh, zbuf, colbuf, wbuf):
    cid = lax.axis_index("c")
    sid = lax.axis_index("s")
    base_g = cid * (NS * K1_GPT) + sid * K1_GPT

    for i in range(RPT // L):
        zbuf[pl.ds(i * L, L)] = _ZV()
    pltpu.sync_copy(zbuf, deg_sh.at[pl.ds(sid * RPT, RPT)])
    pltpu.sync_copy(col_ref.at[pl.ds(base_g, K1_GPT)], colbuf)
    pltpu.sync_copy(w_ref.at[pl.ds(base_g, K1_GPT)], wbuf)
    plsc.subcore_barrier()

    def body(g, carry):
        pltpu.sync_copy(wbuf.at[g], deg_sh.at[colbuf.at[g]], add=True)
        return carry

    lax.fori_loop(0, K1_GPT, body, 0)
    plsc.subcore_barrier()
    pltpu.sync_copy(
        deg_sh.at[pl.ds(sid * RPT, RPT)], deg_out.at[cid, pl.ds(sid * RPT, RPT)]
    )


_RB = 1024


def _dis_body(deg_ref, x_ref, dis_ref, xs_ref):
    dis = lax.rsqrt(deg_ref[0] + deg_ref[1] + 1.0)
    dis_ref[...] = dis
    xs_ref[...] = (dis * x_ref[...]).astype(jnp.bfloat16)


_dis_kernel = pl.pallas_call(
    _dis_body,
    grid=(N_PAD // _RB,),
    in_specs=[
        pl.BlockSpec((NC, _RB, 1), lambda i: (0, i, 0)),
        pl.BlockSpec((_RB, D), lambda i: (i, 0)),
    ],
    out_specs=[
        pl.BlockSpec((_RB, 1), lambda i: (i, 0)),
        pl.BlockSpec((_RB, D), lambda i: (i, 0)),
    ],
    out_shape=[
        jax.ShapeDtypeStruct((N_PAD, 1), jnp.float32),
        jax.ShapeDtypeStruct((N_PAD, D), jnp.bfloat16),
    ],
)


@functools.partial(
    pl.kernel,
    out_type=jax.ShapeDtypeStruct((NC, N_PAD, D), jnp.float32),
    mesh=_sc_mesh,
    scratch_types=[
        pltpu.VMEM_SHARED((N_PAD, D), jnp.float32),
        pltpu.VMEM((NB, G, D // 2), jnp.float32),
        pltpu.VMEM((NS2, G, D), jnp.float32),
        pltpu.VMEM((2, SGC, G), jnp.int32),
        pltpu.VMEM((2, SGC, G), jnp.int32),
        pltpu.VMEM((2, SGC, G), jnp.float32),
        [pltpu.SemaphoreType.DMA] * NB,
        [pltpu.SemaphoreType.DMA] * NS2,
        pltpu.SemaphoreType.DMA,
    ],
    compiler_params=_sc_params_sc_tiling,
)
def _prop_kernel(
    xs_ref, row_ref, col_ref, w_ref, acc_out,
    acc_sh, rows_bf, stage, rowbuf, colbuf, wbuf, gsems, ssems, isem,
):
    cid = lax.axis_index("c")
    sid = lax.axis_index("s")
    base_g = cid * (NS * GPT) + sid * GPT

    def zrow(r, carry):
        for j in range(D // L):
            stage[0, r, pl.ds(j * L, L)] = _ZV()
        return carry

    lax.fori_loop(0, G, zrow, 0)
    for i in range(RPT // G):
        pltpu.sync_copy(stage.at[0], acc_sh.at[pl.ds(sid * RPT + i * G, G)])
    plsc.subcore_barrier()

    def stage_idx(s, slot):
        sg = base_g + s * SGC
        pltpu.async_copy(row_ref.at[pl.ds(sg, SGC)], rowbuf.at[slot], isem)
        pltpu.async_copy(col_ref.at[pl.ds(sg, SGC)], colbuf.at[slot], isem)
        pltpu.async_copy(w_ref.at[pl.ds(sg, SGC)], wbuf.at[slot], isem)

    def stage_idx_wait(s, slot):
        sg = base_g + s * SGC
        pltpu.make_async_copy(row_ref.at[pl.ds(sg, SGC)], rowbuf.at[slot], isem).wait()
        pltpu.make_async_copy(col_ref.at[pl.ds(sg, SGC)], colbuf.at[slot], isem).wait()
        pltpu.make_async_copy(w_ref.at[pl.ds(sg, SGC)], wbuf.at[slot], isem).wait()

    stage_idx(0, 0)
    stage_idx_wait(0, 0)

    def gather(g_local, slot, b):
        pltpu.async_copy(
            xs_ref.at[rowbuf.at[slot, g_local]], rows_bf.at[b], gsems[b]
        )

    def gather_wait(g_local, slot, b):
        pltpu.make_async_copy(
            xs_ref.at[rowbuf.at[slot, g_local]], rows_bf.at[b], gsems[b]
        ).wait()

    def scatter_wait(g_local, slot, c):
        pltpu.make_async_copy(
            stage.at[c], acc_sh.at[colbuf.at[slot, g_local]], ssems[c]
        ).wait()

    M_HI = jnp.int32(-65536)

    def stage_body(s, carry):
        sp = lax.rem(s, 2)

        @pl.when(s > 0)
        def _entry_waits():
            scatter_wait(SGC - 2, 1 - sp, 0)
            scatter_wait(SGC - 1, 1 - sp, 1)
            stage_idx_wait(s, sp)

        @pl.when(s < NSTG - 1)
        def _stage_next():
            stage_idx(s + 1, 1 - sp)

        gather(0, sp, 0)
        gather(1, sp, 1)
        gather(2, sp, 2)

        def ring_body(r, carry2):
            for b in range(NB):
                gl = r * NB + b
                c = b % NS2
                gather_wait(gl, sp, b)

                @pl.when(gl >= 2)
                def _drain():
                    scatter_wait(gl - 2, sp, c)

                def scale16(t, carry3):
                    fvec = wbuf[sp, gl, pl.ds(t * L, L)]
                    for k in range(L):
                        f = fvec[k]
                        e = t * L + k
                        for j in range(D // 32):
                            w32 = plsc.bitcast(
                                rows_bf[b, e, pl.ds(L * j, L)], jnp.int32
                            )
                            lo = plsc.bitcast(w32 << 16, jnp.float32)
                            hi = plsc.bitcast(w32 & M_HI, jnp.float32)
                            stage[c, e, pl.ds(32 * j, L)] = lo * f
                            stage[c, e, pl.ds(32 * j + L, L)] = hi * f
                    return carry3

                lax.fori_loop(0, G // L, scale16, 0)

                pltpu.async_copy(
                    stage.at[c], acc_sh.at[colbuf.at[sp, gl]], ssems[c], add=True
                )

                glp = gl + 3

                @pl.when(glp < SGC)
                def _prefetch():
                    gather(glp, sp, (b + 3) % NB)

            return carry2

        lax.fori_loop(0, SGC // NB, ring_body, 0)
        return carry

    lax.fori_loop(0, NSTG, stage_body, 0)
    scatter_wait(SGC - 2, lax.rem(NSTG - 1, 2), 0)
    scatter_wait(SGC - 1, lax.rem(NSTG - 1, 2), 1)
    plsc.subcore_barrier()
    pltpu.sync_copy(
        acc_sh.at[pl.ds(sid * RPT, RPT)], acc_out.at[cid, pl.ds(sid * RPT, RPT)]
    )


def _epi_body(acc_ref, x_ref, dis_ref, w_ref, b_ref, o_ref):
    d = dis_ref[...]
    t = (acc_ref[0] + acc_ref[1] + d * x_ref[...]) * d
    o_ref[...] = (
        lax.dot_general(
            t, w_ref[...], (((1,), (1,)), ((), ())),
            preferred_element_type=jnp.float32,
        )
        + b_ref[...]
    )


_epi_kernel = pl.pallas_call(
    _epi_body,
    grid=(N_PAD // _RB,),
    in_specs=[
        pl.BlockSpec((NC, _RB, D), lambda i: (0, i, 0)),
        pl.BlockSpec((_RB, D), lambda i: (i, 0)),
        pl.BlockSpec((_RB, 1), lambda i: (i, 0)),
        pl.BlockSpec((D, D), lambda i: (0, 0)),
        pl.BlockSpec((1, D), lambda i: (0, 0)),
    ],
    out_specs=pl.BlockSpec((_RB, D), lambda i: (i, 0)),
    out_shape=jax.ShapeDtypeStruct((N_PAD, D), jnp.float32),
)


def kernel(x, edge_index, edge_weights, W, b):
    row = edge_index[0].astype(jnp.int32)
    col = edge_index[1].astype(jnp.int32)
    w = edge_weights.astype(jnp.float32)

    npad = E_PAD - E
    pad_idx = (jnp.arange(npad, dtype=jnp.int32) * 37) % N_NODES
    row_p = jnp.concatenate([row, pad_idx]).reshape(GROUPS, G)
    col_p = jnp.concatenate([col, pad_idx]).reshape(GROUPS, G)
    w_p = jnp.concatenate([w, jnp.zeros((npad,), jnp.float32)]).reshape(GROUPS, G)

    x_pad = jnp.concatenate(
        [x, jnp.zeros((N_PAD - N_NODES, D), jnp.float32)], axis=0
    )

    deg = _deg_kernel(
        col_p.reshape(E_PAD // K1_G, K1_G), w_p.reshape(E_PAD // K1_G, K1_G)
    )
    dis, xs = _dis_kernel(deg.reshape(NC, N_PAD, 1), x_pad)
    xs_p = xs[:, jnp.asarray(_PERM)]
    xs_pk = lax.bitcast_convert_type(
        xs_p.reshape(N_PAD, D // 2, 2), jnp.float32
    )
    acc = _prop_kernel(xs_pk, row_p, col_p, w_p)
    y = _epi_kernel(acc, x_pad, dis, W, b.reshape(1, D))
    return y[:N_NODES]


# speedup vs baseline: 1.4720x; 1.4720x over previous
import functools

import jax
import jax.numpy as jnp
import numpy as np
from jax import lax
from jax.experimental import pallas as pl
from jax.experimental.pallas import tpu as pltpu
from jax.experimental.pallas import tpu_sc as plsc

N_NODES = 10000
N_PAD = 10240
D = 128
E = 320000
L = 16
NC = 2
NS = 16
G = 64
E_PAD = 327680
GROUPS = E_PAD // G
GPT = GROUPS // (NC * NS)
SGC = 16
NSTG = GPT // SGC
NB = 2
NS2 = 2
RPT = N_PAD // NS
K1_G = 128
K1_GPT = (E_PAD // K1_G) // (NC * NS)

_PERM = np.empty((D,), dtype=np.int32)
for _p in range(D):
    _j, _r = _p // 32, _p % 32
    _PERM[_p] = 32 * _j + (_r // 2) + 16 * (_r % 2)

_sc_mesh = plsc.VectorSubcoreMesh(
    core_axis_name="c", subcore_axis_name="s", num_cores=NC, num_subcores=NS
)

_ZV = lambda: jnp.zeros((L,), jnp.float32)

_sc_params = pltpu.CompilerParams(needs_layout_passes=False)
_sc_params_sc_tiling = pltpu.CompilerParams(
    needs_layout_passes=False, use_tc_tiling_on_sc=False
)


@functools.partial(
    pl.kernel,
    out_type=jax.ShapeDtypeStruct((NC, N_PAD), jnp.float32),
    mesh=_sc_mesh,
    scratch_types=[
        pltpu.VMEM_SHARED((N_PAD,), jnp.float32),
        pltpu.VMEM((RPT,), jnp.float32),
        pltpu.VMEM((K1_GPT, K1_G), jnp.int32),
        pltpu.VMEM((K1_GPT, K1_G), jnp.float32),
    ],
    compiler_params=_sc_params,
)
def _deg_kernel(col_ref, w_ref, deg_out, deg_sh, zbuf, colbuf, wbuf):
    cid = lax.axis_index("c")
    sid = lax.axis_index("s")
    base_g = cid * (NS * K1_GPT) + sid * K1_GPT

    for i in range(RPT // L):
        zbuf[pl.ds(i * L, L)] = _ZV()
    pltpu.sync_copy(zbuf, deg_sh.at[pl.ds(sid * RPT, RPT)])
    pltpu.sync_copy(col_ref.at[pl.ds(base_g, K1_GPT)], colbuf)
    pltpu.sync_copy(w_ref.at[pl.ds(base_g, K1_GPT)], wbuf)
    plsc.subcore_barrier()

    def body(g, carry):
        pltpu.sync_copy(wbuf.at[g], deg_sh.at[colbuf.at[g]], add=True)
        return carry

    lax.fori_loop(0, K1_GPT, body, 0)
    plsc.subcore_barrier()
    pltpu.sync_copy(
        deg_sh.at[pl.ds(sid * RPT, RPT)], deg_out.at[cid, pl.ds(sid * RPT, RPT)]
    )


_RB = 1024


def _dis_body(deg_ref, x_ref, dis_ref, xs_ref):
    dis = lax.rsqrt(deg_ref[0] + deg_ref[1] + 1.0)
    dis_ref[...] = dis
    xs_ref[...] = dis * x_ref[...]


_dis_kernel = pl.pallas_call(
    _dis_body,
    grid=(N_PAD // _RB,),
    in_specs=[
        pl.BlockSpec((NC, _RB, 1), lambda i: (0, i, 0)),
        pl.BlockSpec((_RB, D), lambda i: (i, 0)),
    ],
    out_specs=[
        pl.BlockSpec((_RB, 1), lambda i: (i, 0)),
        pl.BlockSpec((_RB, D), lambda i: (i, 0)),
    ],
    out_shape=[
        jax.ShapeDtypeStruct((N_PAD, 1), jnp.float32),
        jax.ShapeDtypeStruct((N_PAD, D), jnp.float32),
    ],
)


@functools.partial(
    pl.kernel,
    out_type=jax.ShapeDtypeStruct((NC, N_PAD, D), jnp.float32),
    mesh=_sc_mesh,
    scratch_types=[
        pltpu.VMEM_SHARED((N_PAD, D), jnp.float32),
        pltpu.VMEM((NB, G, D), jnp.float32),
        pltpu.VMEM((NS2, G, D), jnp.float32),
        pltpu.VMEM((2, SGC, G), jnp.int32),
        pltpu.VMEM((2, SGC, G), jnp.int32),
        pltpu.VMEM((2, SGC, G), jnp.float32),
        [pltpu.SemaphoreType.DMA] * NB,
        [pltpu.SemaphoreType.DMA] * NS2,
        pltpu.SemaphoreType.DMA,
    ],
    compiler_params=_sc_params,
)
def _prop_kernel(
    xs_ref, row_ref, col_ref, w_ref, acc_out,
    acc_sh, rows_bf, stage, rowbuf, colbuf, wbuf, gsems, ssems, isem,
):
    cid = lax.axis_index("c")
    sid = lax.axis_index("s")
    base_g = cid * (NS * GPT) + sid * GPT

    def zrow(r, carry):
        for j in range(D // L):
            stage[0, r, pl.ds(j * L, L)] = _ZV()
        return carry

    lax.fori_loop(0, G, zrow, 0)
    for i in range(RPT // G):
        pltpu.sync_copy(stage.at[0], acc_sh.at[pl.ds(sid * RPT + i * G, G)])
    plsc.subcore_barrier()

    def stage_idx(s, slot):
        sg = base_g + s * SGC
        pltpu.async_copy(row_ref.at[pl.ds(sg, SGC)], rowbuf.at[slot], isem)
        pltpu.async_copy(col_ref.at[pl.ds(sg, SGC)], colbuf.at[slot], isem)
        pltpu.async_copy(w_ref.at[pl.ds(sg, SGC)], wbuf.at[slot], isem)

    def stage_idx_wait(s, slot):
        sg = base_g + s * SGC
        pltpu.make_async_copy(row_ref.at[pl.ds(sg, SGC)], rowbuf.at[slot], isem).wait()
        pltpu.make_async_copy(col_ref.at[pl.ds(sg, SGC)], colbuf.at[slot], isem).wait()
        pltpu.make_async_copy(w_ref.at[pl.ds(sg, SGC)], wbuf.at[slot], isem).wait()

    stage_idx(0, 0)
    stage_idx_wait(0, 0)

    def gather(g_local, slot, b):
        pltpu.async_copy(
            xs_ref.at[rowbuf.at[slot, g_local]], rows_bf.at[b], gsems[b]
        )

    def gather_wait(g_local, slot, b):
        pltpu.make_async_copy(
            xs_ref.at[rowbuf.at[slot, g_local]], rows_bf.at[b], gsems[b]
        ).wait()

    def scatter_wait(g_local, slot, c):
        pltpu.make_async_copy(
            stage.at[c], acc_sh.at[colbuf.at[slot, g_local]], ssems[c]
        ).wait()

    M_HI = jnp.int32(-65536)

    def stage_body(s, carry):
        sp = lax.rem(s, 2)

        @pl.when(s > 0)
        def _entry_waits():
            scatter_wait(SGC - 1, 1 - sp, 1)
            stage_idx_wait(s, sp)

        @pl.when(s < NSTG - 1)
        def _stage_next():
            stage_idx(s + 1, 1 - sp)

        gather(0, sp, 0)
        gather(1, sp, 1)

        def ring_body(r, carry2):
            for b in range(NB):
                gl = r * NB + b
                c = b % NS2
                gather_wait(gl, sp, b)

                @pl.when(gl >= 1)
                def _drain():
                    scatter_wait(gl - 1, sp, 1 - c)

                def scale16(t, carry3):
                    fvec = wbuf[sp, gl, pl.ds(t * L, L)]
                    for k in range(L):
                        f = fvec[k]
                        e = t * L + k
                        for j in range(D // L):
                            sl = pl.ds(j * L, L)
                            stage[c, e, sl] = rows_bf[b, e, sl] * f
                    return carry3

                lax.fori_loop(0, G // L, scale16, 0)

                pltpu.async_copy(
                    stage.at[c], acc_sh.at[colbuf.at[sp, gl]], ssems[c], add=True
                )

                glp = gl + NB

                @pl.when(glp < SGC)
                def _prefetch():
                    gather(glp, sp, b)

            return carry2

        lax.fori_loop(0, SGC // NB, ring_body, 0)
        return carry

    lax.fori_loop(0, NSTG, stage_body, 0)
    scatter_wait(SGC - 1, lax.rem(NSTG - 1, 2), 1)
    plsc.subcore_barrier()
    pltpu.sync_copy(
        acc_sh.at[pl.ds(sid * RPT, RPT)], acc_out.at[cid, pl.ds(sid * RPT, RPT)]
    )


def _epi_body(acc_ref, x_ref, dis_ref, w_ref, b_ref, o_ref):
    d = dis_ref[...]
    t = (acc_ref[0] + acc_ref[1] + d * x_ref[...]) * d
    o_ref[...] = (
        lax.dot_general(
            t, w_ref[...], (((1,), (1,)), ((), ())),
            preferred_element_type=jnp.float32,
        )
        + b_ref[...]
    )


_epi_kernel = pl.pallas_call(
    _epi_body,
    grid=(N_PAD // _RB,),
    in_specs=[
        pl.BlockSpec((NC, _RB, D), lambda i: (0, i, 0)),
        pl.BlockSpec((_RB, D), lambda i: (i, 0)),
        pl.BlockSpec((_RB, 1), lambda i: (i, 0)),
        pl.BlockSpec((D, D), lambda i: (0, 0)),
        pl.BlockSpec((1, D), lambda i: (0, 0)),
    ],
    out_specs=pl.BlockSpec((_RB, D), lambda i: (i, 0)),
    out_shape=jax.ShapeDtypeStruct((N_PAD, D), jnp.float32),
)


def kernel(x, edge_index, edge_weights, W, b):
    row = edge_index[0].astype(jnp.int32)
    col = edge_index[1].astype(jnp.int32)
    w = edge_weights.astype(jnp.float32)

    npad = E_PAD - E
    pad_idx = (jnp.arange(npad, dtype=jnp.int32) * 37) % N_NODES
    row_p = jnp.concatenate([row, pad_idx]).reshape(GROUPS, G)
    col_p = jnp.concatenate([col, pad_idx]).reshape(GROUPS, G)
    w_p = jnp.concatenate([w, jnp.zeros((npad,), jnp.float32)]).reshape(GROUPS, G)

    x_pad = jnp.concatenate(
        [x, jnp.zeros((N_PAD - N_NODES, D), jnp.float32)], axis=0
    )

    deg = _deg_kernel(
        col_p.reshape(E_PAD // K1_G, K1_G), w_p.reshape(E_PAD // K1_G, K1_G)
    )
    dis, xs = _dis_kernel(deg.reshape(NC, N_PAD, 1), x_pad)
    acc = _prop_kernel(xs, row_p, col_p, w_p)
    y = _epi_kernel(acc, x_pad, dis, W, b.reshape(1, D))
    return y[:N_NODES]


# speedup vs baseline: 1.7751x; 1.2059x over previous
import functools

import jax
import jax.numpy as jnp
import numpy as np
from jax import lax
from jax.experimental import pallas as pl
from jax.experimental.pallas import tpu as pltpu
from jax.experimental.pallas import tpu_sc as plsc

N_NODES = 10000
N_PAD = 10240
D = 128
E = 320000
L = 16
NC = 2
NS = 16
G = 64
E_PAD = 327680
GROUPS = E_PAD // G
GPT = GROUPS // (NC * NS)
SGC = 16
NSTG = GPT // SGC
NB = 4
RPT = N_PAD // NS
K1_G = 128
K1_GPT = (E_PAD // K1_G) // (NC * NS)

_PERM = np.empty((D,), dtype=np.int32)
for _p in range(D):
    _j, _r = _p // 32, _p % 32
    _PERM[_p] = 32 * _j + (_r // 2) + 16 * (_r % 2)

_sc_mesh = plsc.VectorSubcoreMesh(
    core_axis_name="c", subcore_axis_name="s", num_cores=NC, num_subcores=NS
)

_ZV = lambda: jnp.zeros((L,), jnp.float32)

_sc_params = pltpu.CompilerParams(needs_layout_passes=False)
_sc_params_sc_tiling = pltpu.CompilerParams(
    needs_layout_passes=False, use_tc_tiling_on_sc=False
)


@functools.partial(
    pl.kernel,
    out_type=jax.ShapeDtypeStruct((NC, N_PAD), jnp.float32),
    mesh=_sc_mesh,
    scratch_types=[
        pltpu.VMEM_SHARED((N_PAD,), jnp.float32),
        pltpu.VMEM((RPT,), jnp.float32),
        pltpu.VMEM((K1_GPT, K1_G), jnp.int32),
        pltpu.VMEM((K1_GPT, K1_G), jnp.float32),
    ],
    compiler_params=_sc_params,
)
def _deg_kernel(col_ref, w_ref, deg_out, deg_sh, zbuf, colbuf, wbuf):
    cid = lax.axis_index("c")
    sid = lax.axis_index("s")
    base_g = cid * (NS * K1_GPT) + sid * K1_GPT

    for i in range(RPT // L):
        zbuf[pl.ds(i * L, L)] = _ZV()
    pltpu.sync_copy(zbuf, deg_sh.at[pl.ds(sid * RPT, RPT)])
    pltpu.sync_copy(col_ref.at[pl.ds(base_g, K1_GPT)], colbuf)
    pltpu.sync_copy(w_ref.at[pl.ds(base_g, K1_GPT)], wbuf)
    plsc.subcore_barrier()

    def body(g, carry):
        pltpu.sync_copy(wbuf.at[g], deg_sh.at[colbuf.at[g]], add=True)
        return carry

    lax.fori_loop(0, K1_GPT, body, 0)
    plsc.subcore_barrier()
    pltpu.sync_copy(
        deg_sh.at[pl.ds(sid * RPT, RPT)], deg_out.at[cid, pl.ds(sid * RPT, RPT)]
    )


_RB = 1024


def _dis_body(deg_ref, x_ref, dis_ref, xs_ref):
    dis = lax.rsqrt(deg_ref[0] + deg_ref[1] + 1.0)
    dis_ref[...] = dis
    xs_ref[...] = dis * x_ref[...]


_dis_kernel = pl.pallas_call(
    _dis_body,
    grid=(N_PAD // _RB,),
    in_specs=[
        pl.BlockSpec((NC, _RB, 1), lambda i: (0, i, 0)),
        pl.BlockSpec((_RB, D), lambda i: (i, 0)),
    ],
    out_specs=[
        pl.BlockSpec((_RB, 1), lambda i: (i, 0)),
        pl.BlockSpec((_RB, D), lambda i: (i, 0)),
    ],
    out_shape=[
        jax.ShapeDtypeStruct((N_PAD, 1), jnp.float32),
        jax.ShapeDtypeStruct((N_PAD, D), jnp.float32),
    ],
)


@functools.partial(
    pl.kernel,
    out_type=jax.ShapeDtypeStruct((NC, N_PAD, D), jnp.float32),
    mesh=_sc_mesh,
    scratch_types=[
        pltpu.VMEM_SHARED((N_PAD, D), jnp.float32),
        pltpu.VMEM((NB, G, D), jnp.float32),
        pltpu.VMEM((2, SGC, G), jnp.int32),
        pltpu.VMEM((2, SGC, G), jnp.int32),
        pltpu.VMEM((2, SGC, G), jnp.float32),
        [pltpu.SemaphoreType.DMA] * NB,
        [pltpu.SemaphoreType.DMA] * NB,
        pltpu.SemaphoreType.DMA,
    ],
    compiler_params=_sc_params,
)
def _prop_kernel(
    xs_ref, row_ref, col_ref, w_ref, acc_out,
    acc_sh, rows, rowbuf, colbuf, wbuf, gsems, ssems, isem,
):
    cid = lax.axis_index("c")
    sid = lax.axis_index("s")
    base_g = cid * (NS * GPT) + sid * GPT

    def zrow(r, carry):
        for j in range(D // L):
            rows[0, r, pl.ds(j * L, L)] = _ZV()
        return carry

    lax.fori_loop(0, G, zrow, 0)
    for i in range(RPT // G):
        pltpu.sync_copy(rows.at[0], acc_sh.at[pl.ds(sid * RPT + i * G, G)])
    plsc.subcore_barrier()

    def stage_idx(s, slot):
        sg = base_g + s * SGC
        pltpu.async_copy(row_ref.at[pl.ds(sg, SGC)], rowbuf.at[slot], isem)
        pltpu.async_copy(col_ref.at[pl.ds(sg, SGC)], colbuf.at[slot], isem)
        pltpu.async_copy(w_ref.at[pl.ds(sg, SGC)], wbuf.at[slot], isem)

    def stage_idx_wait(s, slot):
        sg = base_g + s * SGC
        pltpu.make_async_copy(row_ref.at[pl.ds(sg, SGC)], rowbuf.at[slot], isem).wait()
        pltpu.make_async_copy(col_ref.at[pl.ds(sg, SGC)], colbuf.at[slot], isem).wait()
        pltpu.make_async_copy(w_ref.at[pl.ds(sg, SGC)], wbuf.at[slot], isem).wait()

    stage_idx(0, 0)
    stage_idx_wait(0, 0)

    def gather(g_local, slot, b):
        pltpu.async_copy(
            xs_ref.at[rowbuf.at[slot, g_local]], rows.at[b], gsems[b]
        )

    def gather_wait(g_local, slot, b):
        pltpu.make_async_copy(
            xs_ref.at[rowbuf.at[slot, g_local]], rows.at[b], gsems[b]
        ).wait()

    def scatter_wait(g_local, slot, b):
        pltpu.make_async_copy(
            rows.at[b], acc_sh.at[colbuf.at[slot, g_local]], ssems[b]
        ).wait()

    def stage_body(s, carry):
        sp = lax.rem(s, 2)

        @pl.when(s > 0)
        def _wait_idx():
            stage_idx_wait(s, sp)

        @pl.when(s < NSTG - 1)
        def _stage_next():
            stage_idx(s + 1, 1 - sp)

        gather(0, sp, 0)
        gather(1, sp, 1)

        def ring_body(r, carry2):
            for b in range(NB):
                gl = r * NB + b
                gather_wait(gl, sp, b)

                def scale16(t, carry3):
                    fvec = wbuf[sp, gl, pl.ds(t * L, L)]
                    for k in range(L):
                        f = fvec[k]
                        e = t * L + k
                        for j in range(D // L):
                            sl = pl.ds(j * L, L)
                            rows[b, e, sl] = rows[b, e, sl] * f
                    return carry3

                lax.fori_loop(0, G // L, scale16, 0)

                pltpu.async_copy(
                    rows.at[b], acc_sh.at[colbuf.at[sp, gl]], ssems[b], add=True
                )

                bp = (b + 2) % NB
                glp = gl + 2

                @pl.when(glp < SGC)
                def _prefetch():
                    @pl.when(gl >= 2)
                    def _drain():
                        scatter_wait(glp - NB, sp, bp)

                    gather(glp, sp, bp)

            return carry2

        lax.fori_loop(0, SGC // NB, ring_body, 0)

        for b in range(NB):
            scatter_wait(SGC - NB + b, sp, b)
        return carry

    lax.fori_loop(0, NSTG, stage_body, 0)
    plsc.subcore_barrier()
    pltpu.sync_copy(
        acc_sh.at[pl.ds(sid * RPT, RPT)], acc_out.at[cid, pl.ds(sid * RPT, RPT)]
    )


def _epi_body(acc_ref, x_ref, dis_ref, w_ref, b_ref, o_ref):
    d = dis_ref[...]
    t = (acc_ref[0] + acc_ref[1] + d * x_ref[...]) * d
    o_ref[...] = (
        lax.dot_general(
            t, w_ref[...], (((1,), (1,)), ((), ())),
            preferred_element_type=jnp.float32,
        )
        + b_ref[...]
    )


_epi_kernel = pl.pallas_call(
    _epi_body,
    grid=(N_PAD // _RB,),
    in_specs=[
        pl.BlockSpec((NC, _RB, D), lambda i: (0, i, 0)),
        pl.BlockSpec((_RB, D), lambda i: (i, 0)),
        pl.BlockSpec((_RB, 1), lambda i: (i, 0)),
        pl.BlockSpec((D, D), lambda i: (0, 0)),
        pl.BlockSpec((1, D), lambda i: (0, 0)),
    ],
    out_specs=pl.BlockSpec((_RB, D), lambda i: (i, 0)),
    out_shape=jax.ShapeDtypeStruct((N_PAD, D), jnp.float32),
)


def kernel(x, edge_index, edge_weights, W, b):
    row = edge_index[0].astype(jnp.int32)
    col = edge_index[1].astype(jnp.int32)
    w = edge_weights.astype(jnp.float32)

    npad = E_PAD - E
    pad_idx = (jnp.arange(npad, dtype=jnp.int32) * 37) % N_NODES
    row_p = jnp.concatenate([row, pad_idx]).reshape(GROUPS, G)
    col_p = jnp.concatenate([col, pad_idx]).reshape(GROUPS, G)
    w_p = jnp.concatenate([w, jnp.zeros((npad,), jnp.float32)]).reshape(GROUPS, G)

    x_pad = jnp.concatenate(
        [x, jnp.zeros((N_PAD - N_NODES, D), jnp.float32)], axis=0
    )

    deg = _deg_kernel(
        col_p.reshape(E_PAD // K1_G, K1_G), w_p.reshape(E_PAD // K1_G, K1_G)
    )
    dis, xs = _dis_kernel(deg.reshape(NC, N_PAD, 1), x_pad)
    acc = _prop_kernel(xs, row_p, col_p, w_p)
    y = _epi_kernel(acc, x_pad, dis, W, b.reshape(1, D))
    return y[:N_NODES]


# speedup vs baseline: 1.8053x; 1.0170x over previous
import functools

import jax
import jax.numpy as jnp
import numpy as np
from jax import lax
from jax.experimental import pallas as pl
from jax.experimental.pallas import tpu as pltpu
from jax.experimental.pallas import tpu_sc as plsc

N_NODES = 10000
N_PAD = 10240
D = 128
E = 320000
L = 16
NC = 2
NS = 16
G = 64
E_PAD = 327680
GROUPS = E_PAD // G
GPT = GROUPS // (NC * NS)
SGC = 16
NSTG = GPT // SGC
NB = 4
RPT = N_PAD // NS
K1_G = 128
K1_GPT = (E_PAD // K1_G) // (NC * NS)

_PERM = np.empty((D,), dtype=np.int32)
for _p in range(D):
    _j, _r = _p // 32, _p % 32
    _PERM[_p] = 32 * _j + (_r // 2) + 16 * (_r % 2)

_sc_mesh = plsc.VectorSubcoreMesh(
    core_axis_name="c", subcore_axis_name="s", num_cores=NC, num_subcores=NS
)

_ZV = lambda: jnp.zeros((L,), jnp.float32)

_sc_params = pltpu.CompilerParams(needs_layout_passes=False)
_sc_params_sc_tiling = pltpu.CompilerParams(
    needs_layout_passes=False, use_tc_tiling_on_sc=False
)


@functools.partial(
    pl.kernel,
    out_type=jax.ShapeDtypeStruct((NC, N_PAD), jnp.float32),
    mesh=_sc_mesh,
    scratch_types=[
        pltpu.VMEM_SHARED((N_PAD,), jnp.float32),
        pltpu.VMEM((RPT,), jnp.float32),
        pltpu.VMEM((K1_GPT, K1_G), jnp.int32),
        pltpu.VMEM((K1_GPT, K1_G), jnp.float32),
    ],
    compiler_params=_sc_params,
)
def _deg_kernel(col_ref, w_ref, deg_out, deg_sh, zbuf, colbuf, wbuf):
    cid = lax.axis_index("c")
    sid = lax.axis_index("s")
    base_g = cid * (NS * K1_GPT) + sid * K1_GPT

    for i in range(RPT // L):
        zbuf[pl.ds(i * L, L)] = _ZV()
    pltpu.sync_copy(zbuf, deg_sh.at[pl.ds(sid * RPT, RPT)])
    pltpu.sync_copy(col_ref.at[pl.ds(base_g, K1_GPT)], colbuf)
    pltpu.sync_copy(w_ref.at[pl.ds(base_g, K1_GPT)], wbuf)
    plsc.subcore_barrier()

    def body(g, carry):
        pltpu.sync_copy(wbuf.at[g], deg_sh.at[colbuf.at[g]], add=True)
        return carry

    lax.fori_loop(0, K1_GPT, body, 0)
    plsc.subcore_barrier()
    pltpu.sync_copy(
        deg_sh.at[pl.ds(sid * RPT, RPT)], deg_out.at[cid, pl.ds(sid * RPT, RPT)]
    )


_RB = 1000


def _dis_body(deg_ref, x_ref, dis_ref, xs_ref):
    dis = lax.rsqrt(deg_ref[0] + deg_ref[1] + 1.0)
    dis_ref[...] = dis
    xs_ref[...] = dis * x_ref[...]


_dis_kernel = pl.pallas_call(
    _dis_body,
    grid=(N_NODES // _RB,),
    in_specs=[
        pl.BlockSpec((NC, _RB, 1), lambda i: (0, i, 0)),
        pl.BlockSpec((_RB, D), lambda i: (i, 0)),
    ],
    out_specs=[
        pl.BlockSpec((_RB, 1), lambda i: (i, 0)),
        pl.BlockSpec((_RB, D), lambda i: (i, 0)),
    ],
    out_shape=[
        jax.ShapeDtypeStruct((N_PAD, 1), jnp.float32),
        jax.ShapeDtypeStruct((N_PAD, D), jnp.float32),
    ],
)


@functools.partial(
    pl.kernel,
    out_type=jax.ShapeDtypeStruct((NC, N_PAD, D), jnp.float32),
    mesh=_sc_mesh,
    scratch_types=[
        pltpu.VMEM_SHARED((N_PAD, D), jnp.float32),
        pltpu.VMEM((NB, G, D), jnp.float32),
        pltpu.VMEM((2, SGC, G), jnp.int32),
        pltpu.VMEM((2, SGC, G), jnp.int32),
        pltpu.VMEM((2, SGC, G), jnp.float32),
        [pltpu.SemaphoreType.DMA] * NB,
        [pltpu.SemaphoreType.DMA] * NB,
        pltpu.SemaphoreType.DMA,
    ],
    compiler_params=_sc_params,
)
def _prop_kernel(
    xs_ref, row_ref, col_ref, w_ref, acc_out,
    acc_sh, rows, rowbuf, colbuf, wbuf, gsems, ssems, isem,
):
    cid = lax.axis_index("c")
    sid = lax.axis_index("s")
    base_g = cid * (NS * GPT) + sid * GPT

    def zrow(r, carry):
        for j in range(D // L):
            rows[0, r, pl.ds(j * L, L)] = _ZV()
        return carry

    lax.fori_loop(0, G, zrow, 0)
    for i in range(RPT // G):
        pltpu.sync_copy(rows.at[0], acc_sh.at[pl.ds(sid * RPT + i * G, G)])
    plsc.subcore_barrier()

    def stage_idx(s, slot):
        sg = base_g + s * SGC
        pltpu.async_copy(row_ref.at[pl.ds(sg, SGC)], rowbuf.at[slot], isem)
        pltpu.async_copy(col_ref.at[pl.ds(sg, SGC)], colbuf.at[slot], isem)
        pltpu.async_copy(w_ref.at[pl.ds(sg, SGC)], wbuf.at[slot], isem)

    def stage_idx_wait(s, slot):
        sg = base_g + s * SGC
        pltpu.make_async_copy(row_ref.at[pl.ds(sg, SGC)], rowbuf.at[slot], isem).wait()
        pltpu.make_async_copy(col_ref.at[pl.ds(sg, SGC)], colbuf.at[slot], isem).wait()
        pltpu.make_async_copy(w_ref.at[pl.ds(sg, SGC)], wbuf.at[slot], isem).wait()

    stage_idx(0, 0)
    stage_idx_wait(0, 0)

    def gather(g_local, slot, b):
        pltpu.async_copy(
            xs_ref.at[rowbuf.at[slot, g_local]], rows.at[b], gsems[b]
        )

    def gather_wait(g_local, slot, b):
        pltpu.make_async_copy(
            xs_ref.at[rowbuf.at[slot, g_local]], rows.at[b], gsems[b]
        ).wait()

    def scatter_wait(g_local, slot, b):
        pltpu.make_async_copy(
            rows.at[b], acc_sh.at[colbuf.at[slot, g_local]], ssems[b]
        ).wait()

    def stage_body(s, carry):
        sp = lax.rem(s, 2)

        @pl.when(s > 0)
        def _wait_idx():
            stage_idx_wait(s, sp)

        @pl.when(s < NSTG - 1)
        def _stage_next():
            stage_idx(s + 1, 1 - sp)

        gather(0, sp, 0)
        gather(1, sp, 1)

        def ring_body(r, carry2):
            for b in range(NB):
                gl = r * NB + b
                gather_wait(gl, sp, b)

                def scale16(t, carry3):
                    fvec = wbuf[sp, gl, pl.ds(t * L, L)]
                    for k in range(L):
                        f = fvec[k]
                        e = t * L + k
                        for j in range(D // L):
                            sl = pl.ds(j * L, L)
                            rows[b, e, sl] = rows[b, e, sl] * f
                    return carry3

                lax.fori_loop(0, G // L, scale16, 0)

                pltpu.async_copy(
                    rows.at[b], acc_sh.at[colbuf.at[sp, gl]], ssems[b], add=True
                )

                bp = (b + 2) % NB
                glp = gl + 2

                @pl.when(glp < SGC)
                def _prefetch():
                    @pl.when(gl >= 2)
                    def _drain():
                        scatter_wait(glp - NB, sp, bp)

                    gather(glp, sp, bp)

            return carry2

        lax.fori_loop(0, SGC // NB, ring_body, 0)

        for b in range(NB):
            scatter_wait(SGC - NB + b, sp, b)
        return carry

    lax.fori_loop(0, NSTG, stage_body, 0)
    plsc.subcore_barrier()
    pltpu.sync_copy(
        acc_sh.at[pl.ds(sid * RPT, RPT)], acc_out.at[cid, pl.ds(sid * RPT, RPT)]
    )


def _epi_body(acc_ref, x_ref, dis_ref, w_ref, b_ref, o_ref):
    d = dis_ref[...]
    t = (acc_ref[0] + acc_ref[1] + d * x_ref[...]) * d
    o_ref[...] = (
        lax.dot_general(
            t, w_ref[...], (((1,), (1,)), ((), ())),
            preferred_element_type=jnp.float32,
        )
        + b_ref[...]
    )


_epi_kernel = pl.pallas_call(
    _epi_body,
    grid=(N_NODES // _RB,),
    in_specs=[
        pl.BlockSpec((NC, _RB, D), lambda i: (0, i, 0)),
        pl.BlockSpec((_RB, D), lambda i: (i, 0)),
        pl.BlockSpec((_RB, 1), lambda i: (i, 0)),
        pl.BlockSpec((D, D), lambda i: (0, 0)),
        pl.BlockSpec((1, D), lambda i: (0, 0)),
    ],
    out_specs=pl.BlockSpec((_RB, D), lambda i: (i, 0)),
    out_shape=jax.ShapeDtypeStruct((N_NODES, D), jnp.float32),
)


def kernel(x, edge_index, edge_weights, W, b):
    row = edge_index[0].astype(jnp.int32)
    col = edge_index[1].astype(jnp.int32)
    w = edge_weights.astype(jnp.float32)

    npad = E_PAD - E
    pad_idx = (jnp.arange(npad, dtype=jnp.int32) * 37) % N_NODES
    row_p = jnp.concatenate([row, pad_idx]).reshape(GROUPS, G)
    col_p = jnp.concatenate([col, pad_idx]).reshape(GROUPS, G)
    w_p = jnp.concatenate([w, jnp.zeros((npad,), jnp.float32)]).reshape(GROUPS, G)

    deg = _deg_kernel(
        col_p.reshape(E_PAD // K1_G, K1_G), w_p.reshape(E_PAD // K1_G, K1_G)
    )
    dis, xs = _dis_kernel(deg.reshape(NC, N_PAD, 1), x)
    acc = _prop_kernel(xs, row_p, col_p, w_p)
    return _epi_kernel(acc, x, dis, W, b.reshape(1, D))
